# baseline (speedup 1.0000x reference)
import functools

import jax
import jax.numpy as jnp
from jax import lax
from jax.experimental import pallas as pl
from jax.experimental.pallas import tpu as pltpu
from jax.experimental.pallas import tpu_sc as plsc

N_BITS = 16
NUM_SAMPLES = 1_000_000
NUM_WORKERS = 32
ROWS_PER_W = NUM_SAMPLES // NUM_WORKERS
CHUNK = 3125
NCHUNK = ROWS_PER_W // CHUNK
UNROLL = 5
INNER = CHUNK // UNROLL
CW = CHUNK * N_BITS

_mesh = plsc.VectorSubcoreMesh(core_axis_name="c", subcore_axis_name="s")


@functools.partial(
    pl.kernel,
    out_type=jax.ShapeDtypeStruct((NUM_WORKERS, N_BITS), jnp.float32),
    mesh=_mesh,
    scratch_types=[
        pltpu.VMEM((2 * CW,), jnp.float32),
        pltpu.VMEM((N_BITS,), jnp.float32),
        pltpu.VMEM((N_BITS,), jnp.float32),
        pltpu.SemaphoreType.DMA,
        pltpu.SemaphoreType.DMA,
    ],
)
def _bitsum_sc(logits_hbm, uniforms_hbm, partials_hbm, buf, lv, av, sem0, sem1):
    wid = lax.axis_index("s") * 2 + lax.axis_index("c")
    base = pl.multiple_of(wid * (ROWS_PER_W * N_BITS), N_BITS)

    pltpu.sync_copy(logits_hbm, lv)
    probs = 1.0 / (1.0 + jnp.exp(-lv[...]))

    sems = (sem0, sem1)

    def start(g):
        b = g % 2
        return pltpu.async_copy(
            uniforms_hbm.at[pl.ds(base + g * CW, CW)],
            buf.at[pl.ds(b * CW, CW)],
            sems[b],
        )

    accs = tuple(jnp.zeros((N_BITS,), jnp.float32) for _ in range(UNROLL))
    pending = start(0)
    for g in range(NCHUNK):
        nxt = start(g + 1) if g + 1 < NCHUNK else None
        pending.wait()
        buf0 = (g % 2) * CW

        def body(i, accs, buf0=buf0):
            out = []
            woff = pl.multiple_of(buf0 + i * (UNROLL * N_BITS), N_BITS)
            for k in range(UNROLL):
                u = buf[pl.ds(woff + k * N_BITS, N_BITS)]
                out.append(accs[k] + jnp.where(u < probs, 1.0, 0.0))
            return tuple(out)

        accs = lax.fori_loop(0, INNER, body, accs)
        pending = nxt

    total = accs[0]
    for a in accs[1:]:
        total = total + a
    av[...] = total
    pltpu.sync_copy(av, partials_hbm.at[wid])


def _combine_tc(partials_ref, ops_ref, out_ref):
    total = jnp.sum(partials_ref[...], axis=0)
    m = 1.0 - (2.0 / NUM_SAMPLES) * total
    out_ref[...] = jnp.sum(ops_ref[...] * m[None, :], axis=1)[None, :]


def kernel(logits, uniforms, ops):
    flat = jnp.reshape(uniforms, (NUM_SAMPLES * N_BITS,))
    partials = _bitsum_sc(logits, flat)
    out = pl.pallas_call(
        _combine_tc,
        out_shape=jax.ShapeDtypeStruct((1, ops.shape[0]), jnp.float32),
    )(partials, ops)
    return out[0]


# speedup vs baseline: 1.0314x; 1.0314x over previous
import functools

import jax
import jax.numpy as jnp
from jax import lax
from jax.experimental import pallas as pl
from jax.experimental.pallas import tpu as pltpu
from jax.experimental.pallas import tpu_sc as plsc

N_BITS = 16
NUM_SAMPLES = 1_000_000
NUM_WORKERS = 32
ROWS_PER_W = 31248
REM_ROWS = NUM_SAMPLES - NUM_WORKERS * ROWS_PER_W
CHUNK = 504
NCHUNK = ROWS_PER_W // CHUNK
UNROLL = 4
INNER = CHUNK // UNROLL

_mesh = plsc.VectorSubcoreMesh(core_axis_name="c", subcore_axis_name="s")


@functools.partial(
    pl.kernel,
    out_type=jax.ShapeDtypeStruct((NUM_WORKERS, N_BITS), jnp.float32),
    mesh=_mesh,
    scratch_types=[
        pltpu.VMEM((2 * CHUNK, N_BITS), jnp.float32),
        pltpu.VMEM((N_BITS,), jnp.float32),
        pltpu.VMEM((N_BITS,), jnp.float32),
        pltpu.SemaphoreType.DMA,
        pltpu.SemaphoreType.DMA,
    ],
)
def _bitsum_sc(logits_hbm, uniforms_hbm, partials_hbm, buf, lv, av, sem0, sem1):
    wid = lax.axis_index("s") * 2 + lax.axis_index("c")
    base = pl.multiple_of(wid * ROWS_PER_W, 8)

    pltpu.sync_copy(logits_hbm, lv)
    probs = 1.0 / (1.0 + jnp.exp(-lv[...]))

    sems = (sem0, sem1)

    def start(g, b):
        row = pl.multiple_of(base + g * CHUNK, 8)
        return pltpu.async_copy(
            uniforms_hbm.at[pl.ds(row, CHUNK), :],
            buf.at[pl.ds(b * CHUNK, CHUNK), :],
            sems[b],
        )

    def chunk_reduce(b, accs):
        row0 = b * CHUNK

        def body(i, accs):
            out = []
            r = row0 + i * UNROLL
            for k in range(UNROLL):
                u = buf[r + k, :]
                out.append(accs[k] + jnp.where(u < probs, 1.0, 0.0))
            return tuple(out)

        return lax.fori_loop(0, INNER, body, accs)

    accs = tuple(jnp.zeros((N_BITS,), jnp.float32) for _ in range(UNROLL))
    start(0, 0)
    start(1, 1)

    def wait_for(b):
        pltpu.make_async_copy(
            uniforms_hbm.at[pl.ds(0, CHUNK), :],
            buf.at[pl.ds(b * CHUNK, CHUNK), :],
            sems[b],
        ).wait()

    def step(i, accs):
        g = 2 * i
        wait_for(0)
        accs = chunk_reduce(0, accs)
        start(g + 2, 0)
        wait_for(1)
        accs = chunk_reduce(1, accs)
        start(g + 3, 1)
        return accs

    accs = lax.fori_loop(0, NCHUNK // 2 - 1, step, accs)
    wait_for(0)
    accs = chunk_reduce(0, accs)
    wait_for(1)
    accs = chunk_reduce(1, accs)

    total = accs[0]
    for a in accs[1:]:
        total = total + a
    av[...] = total

    @pl.when(wid == NUM_WORKERS - 1)
    def _():
        pltpu.async_copy(
            uniforms_hbm.at[pl.ds(NUM_WORKERS * ROWS_PER_W, REM_ROWS), :],
            buf.at[pl.ds(0, REM_ROWS), :],
            sem0,
        ).wait()

        def ebody(i, acc):
            u = buf[i, :]
            return acc + jnp.where(u < probs, 1.0, 0.0)

        extra = lax.fori_loop(0, REM_ROWS, ebody, jnp.zeros((N_BITS,), jnp.float32))
        av[...] = av[...] + extra

    pltpu.sync_copy(av, partials_hbm.at[wid])


def _combine_tc(partials_ref, ops_ref, out_ref):
    total = jnp.sum(partials_ref[...], axis=0)
    m = 1.0 - (2.0 / NUM_SAMPLES) * total
    out_ref[...] = jnp.sum(ops_ref[...] * m[None, :], axis=1)[None, :]


def kernel(logits, uniforms, ops):
    partials = _bitsum_sc(logits, uniforms)
    out = pl.pallas_call(
        _combine_tc,
        out_shape=jax.ShapeDtypeStruct((1, ops.shape[0]), jnp.float32),
    )(partials, ops)
    return out[0]


# speedup vs baseline: 9.3156x; 9.0317x over previous
import functools

import jax
import jax.numpy as jnp
from jax import lax
from jax.experimental import pallas as pl
from jax.experimental.pallas import tpu as pltpu
from jax.experimental.pallas import tpu_sc as plsc

N_BITS = 16
NUM_SAMPLES = 1_000_000
NUM_WORKERS = 32
W = 2560
NCH = 12
COLS_PER_W = W * NCH
COLS_ALL = NUM_WORKERS * COLS_PER_W
TAIL = NUM_SAMPLES - COLS_ALL
UNROLL = 8
INNER = W // (16 * UNROLL)

_mesh = plsc.VectorSubcoreMesh(core_axis_name="c", subcore_axis_name="s")


@functools.partial(
    pl.kernel,
    out_type=jax.ShapeDtypeStruct((NUM_WORKERS, N_BITS, 16), jnp.float32),
    mesh=_mesh,
    scratch_types=[
        pltpu.VMEM((2, N_BITS, W), jnp.float32),
        pltpu.VMEM((N_BITS, 16), jnp.float32),
        pltpu.VMEM((N_BITS, 16), jnp.float32),
        pltpu.SemaphoreType.DMA,
        pltpu.SemaphoreType.DMA,
    ],
)
def _bitsum_sc(pmat_hbm, ut_hbm, partials_hbm, buf, pmv, avv, sem0, sem1):
    wid = lax.axis_index("s") * 2 + lax.axis_index("c")
    base = pl.multiple_of(wid * COLS_PER_W, 128)

    pltpu.sync_copy(pmat_hbm, pmv)

    sems = (sem0, sem1)

    def start(t, b):
        col = pl.multiple_of(base + t * W, 128)
        return pltpu.async_copy(
            ut_hbm.at[:, pl.ds(col, W)],
            buf.at[b],
            sems[b],
        )

    def wait_for(b):
        pltpu.make_async_copy(
            ut_hbm.at[:, pl.ds(0, W)],
            buf.at[b],
            sems[b],
        ).wait()

    def init_body(i, c):
        avv[i, :] = jnp.zeros((16,), jnp.float32)
        return c

    lax.fori_loop(0, N_BITS, init_body, 0)

    def process(b):
        def bit_body(i, c):
            pv = pmv[i, :]

            def jbody(j, ts):
                col0 = j * (16 * UNROLL)
                out = []
                for k in range(UNROLL):
                    u = buf[b, i, pl.ds(col0 + k * 16, 16)]
                    out.append(ts[k] + jnp.where(u < pv, 1.0, 0.0))
                return tuple(out)

            ts = lax.fori_loop(
                0, INNER, jbody,
                tuple(jnp.zeros((16,), jnp.float32) for _ in range(UNROLL)),
            )
            s = ts[0]
            for t in ts[1:]:
                s = s + t
            avv[i, :] = avv[i, :] + s
            return c

        lax.fori_loop(0, N_BITS, bit_body, 0)

    start(0, 0)
    start(1, 1)

    def step(t, c):
        wait_for(0)
        process(0)
        start(2 * t + 2, 0)
        wait_for(1)
        process(1)
        start(2 * t + 3, 1)
        return c

    lax.fori_loop(0, NCH // 2 - 1, step, 0)
    wait_for(0)
    process(0)
    wait_for(1)
    process(1)

    pltpu.sync_copy(avv, partials_hbm.at[wid])


def _combine_tc(partials_ref, ops_ref, tail_ref, logits_ref, out_ref):
    probs = 1.0 / (1.0 + jnp.exp(-logits_ref[...]))
    tcnt = jnp.sum(
        jnp.where(tail_ref[...] < probs[:, None], 1.0, 0.0), axis=1
    )
    total = jnp.sum(partials_ref[...], axis=(0, 2)) + tcnt
    m = 1.0 - (2.0 / NUM_SAMPLES) * total
    out_ref[...] = jnp.sum(ops_ref[...] * m[None, :], axis=1)[None, :]


def kernel(logits, uniforms, ops):
    ut = jnp.transpose(uniforms)
    probs = 1.0 / (1.0 + jnp.exp(-logits))
    pmat = jnp.broadcast_to(probs[:, None], (N_BITS, 16))
    partials = _bitsum_sc(pmat, ut)
    tail = lax.slice(ut, (0, COLS_ALL), (N_BITS, NUM_SAMPLES))
    out = pl.pallas_call(
        _combine_tc,
        out_shape=jax.ShapeDtypeStruct((1, ops.shape[0]), jnp.float32),
    )(partials, ops, tail, logits)
    return out[0]


# speedup vs baseline: 10.8740x; 1.1673x over previous
import functools

import jax
import jax.numpy as jnp
from jax import lax
from jax.experimental import pallas as pl
from jax.experimental.pallas import tpu as pltpu
from jax.experimental.pallas import tpu_sc as plsc

N_BITS = 16
NUM_SAMPLES = 1_000_000
NUM_WORKERS = 32
W = 2560
NCH = 6
COLS_PER_W = W * NCH
SC_COLS = NUM_WORKERS * COLS_PER_W
BLK = 32768
NBLK = 15
TC_COLS = BLK * NBLK
TAIL = NUM_SAMPLES - SC_COLS - TC_COLS
UNROLL = 8
INNER = W // (16 * UNROLL)

_mesh = plsc.VectorSubcoreMesh(core_axis_name="c", subcore_axis_name="s")


@functools.partial(
    pl.kernel,
    out_type=jax.ShapeDtypeStruct((NUM_WORKERS, N_BITS, 16), jnp.float32),
    mesh=_mesh,
    scratch_types=[
        pltpu.VMEM((2, N_BITS, W), jnp.float32),
        pltpu.VMEM((N_BITS, 16), jnp.float32),
        pltpu.VMEM((N_BITS, 16), jnp.float32),
        pltpu.SemaphoreType.DMA,
        pltpu.SemaphoreType.DMA,
    ],
)
def _bitsum_sc(pmat_hbm, ut_hbm, partials_hbm, buf, pmv, avv, sem0, sem1):
    wid = lax.axis_index("s") * 2 + lax.axis_index("c")
    base = pl.multiple_of(wid * COLS_PER_W, 128)

    pltpu.sync_copy(pmat_hbm, pmv)

    sems = (sem0, sem1)

    def start(t, b):
        col = pl.multiple_of(base + t * W, 128)
        return pltpu.async_copy(
            ut_hbm.at[:, pl.ds(col, W)],
            buf.at[b],
            sems[b],
        )

    def wait_for(b):
        pltpu.make_async_copy(
            ut_hbm.at[:, pl.ds(0, W)],
            buf.at[b],
            sems[b],
        ).wait()

    def init_body(i, c):
        avv[i, :] = jnp.zeros((16,), jnp.float32)
        return c

    lax.fori_loop(0, N_BITS, init_body, 0)

    def process(b):
        def bit_body(i, c):
            pv = pmv[i, :]

            def jbody(j, ts):
                col0 = j * (16 * UNROLL)
                out = []
                for k in range(UNROLL):
                    u = buf[b, i, pl.ds(col0 + k * 16, 16)]
                    out.append(ts[k] + jnp.where(u < pv, 1.0, 0.0))
                return tuple(out)

            ts = lax.fori_loop(
                0, INNER, jbody,
                tuple(jnp.zeros((16,), jnp.float32) for _ in range(UNROLL)),
            )
            s = ts[0]
            for t in ts[1:]:
                s = s + t
            avv[i, :] = avv[i, :] + s
            return c

        lax.fori_loop(0, N_BITS, bit_body, 0)

    start(0, 0)
    start(1, 1)

    def step(t, c):
        wait_for(0)
        process(0)
        start(2 * t + 2, 0)
        wait_for(1)
        process(1)
        start(2 * t + 3, 1)
        return c

    lax.fori_loop(0, NCH // 2 - 1, step, 0)
    wait_for(0)
    process(0)
    wait_for(1)
    process(1)

    pltpu.sync_copy(avv, partials_hbm.at[wid])


def _tcount_tc(pmat_ref, ut_ref, out_ref):
    pv = pmat_ref[...][:, 0:1]
    cnt = jnp.sum(
        jnp.where(ut_ref[...] < pv, 1.0, 0.0), axis=1, keepdims=True
    )
    out_ref[...] = jnp.broadcast_to(cnt, (N_BITS, 128))


def _combine_tc(partials_ref, tcc_ref, ops_ref, tail_ref, logits_ref, out_ref):
    probs = 1.0 / (1.0 + jnp.exp(-logits_ref[...]))
    tailcnt = jnp.sum(
        jnp.where(tail_ref[...] < probs[:, None], 1.0, 0.0), axis=1
    )
    total = (
        jnp.sum(partials_ref[...], axis=(0, 2))
        + jnp.sum(tcc_ref[...], axis=1) * (1.0 / 128.0)
        + tailcnt
    )
    m = 1.0 - (2.0 / NUM_SAMPLES) * total
    out_ref[...] = jnp.sum(ops_ref[...] * m[None, :], axis=1)[None, :]


def kernel(logits, uniforms, ops):
    ut = jnp.transpose(uniforms)
    probs = 1.0 / (1.0 + jnp.exp(-logits))
    pmat = jnp.broadcast_to(probs[:, None], (N_BITS, 16))

    partials = _bitsum_sc(pmat, ut)

    tcc = pl.pallas_call(
        _tcount_tc,
        grid=(NBLK,),
        in_specs=[
            pl.BlockSpec((N_BITS, 16), lambda g: (0, 0)),
            pl.BlockSpec((N_BITS, BLK), lambda g: (0, g + SC_COLS // BLK)),
        ],
        out_specs=pl.BlockSpec((N_BITS, 128), lambda g: (0, g)),
        out_shape=jax.ShapeDtypeStruct((N_BITS, NBLK * 128), jnp.float32),
    )(pmat, ut)

    tail = lax.slice(ut, (0, SC_COLS + TC_COLS), (N_BITS, NUM_SAMPLES))
    out = pl.pallas_call(
        _combine_tc,
        out_shape=jax.ShapeDtypeStruct((1, ops.shape[0]), jnp.float32),
    )(partials, tcc, ops, tail, logits)
    return out[0]
